# Initial kernel scaffold; baseline (speedup 1.0000x reference)
import jax
import jax.numpy as jnp
from jax.experimental import pallas as pl


def kernel(x, table):
    raise NotImplementedError("write your pallas kernel here")


# speedup vs baseline: 1.4594x; 1.4594x over previous
import functools
import math

import jax
import jax.numpy as jnp
from jax import lax
from jax.experimental import pallas as pl
from jax.experimental.pallas import tpu as pltpu
from jax.experimental.pallas import tpu_sc as plsc

_DIM = 32
_B = 4096 * 200
_NW = 32
_BPW = _B // _NW
_K = 8
_C = _K * 128
_NCHUNK = _BPW // _C
_XROWS = _B // 128
_SCALE = math.sqrt(_DIM)

_mesh = plsc.VectorSubcoreMesh(core_axis_name="c", subcore_axis_name="s")


@functools.partial(
    pl.kernel,
    out_type=jax.ShapeDtypeStruct((_B, _DIM), jnp.float32),
    mesh=_mesh,
    compiler_params=pltpu.CompilerParams(use_tc_tiling_on_sc=False),
    scratch_types=[
        pltpu.VMEM((_K, 128), jnp.int32),
        pltpu.VMEM((_K, 128), jnp.int32),
        pltpu.VMEM((_C, _DIM), jnp.float32),
        pltpu.VMEM((_C, _DIM), jnp.float32),
        pltpu.SemaphoreType.DMA,
        pltpu.SemaphoreType.DMA,
        pltpu.SemaphoreType.DMA,
        pltpu.SemaphoreType.DMA,
    ],
)
def _sc_embed(x_hbm, table_hbm, out_hbm, ibuf0, ibuf1, buf0, buf1,
              sg0, sg1, ss0, ss1):
    wid = lax.axis_index("s") * 2 + lax.axis_index("c")
    row_base = wid * (_BPW // 128)
    base = wid * _BPW
    ibufs = (ibuf0, ibuf1)
    bufs = (buf0, buf1)
    sgs = (sg0, sg1)
    sss = (ss0, ss1)

    def start_gather(g, b):
        pltpu.sync_copy(x_hbm.at[pl.ds(row_base + g * _K, _K)], ibufs[b])
        for j in range(_K):
            pltpu.async_copy(table_hbm.at[ibufs[b].at[j]],
                             bufs[b].at[pl.ds(j * 128, 128)], sgs[b])

    def wait_gather(b):
        for j in range(_K):
            pltpu.make_async_copy(table_hbm.at[ibufs[b].at[j]],
                                  bufs[b].at[pl.ds(j * 128, 128)],
                                  sgs[b]).wait()

    def scale_chunk(buf):
        @plsc.parallel_loop(0, _C, unroll=8)
        def _(r):
            buf[r, pl.ds(0, 16)] = buf[r, pl.ds(0, 16)] * _SCALE
            buf[r, pl.ds(16, 16)] = buf[r, pl.ds(16, 16)] * _SCALE

    start_gather(0, 0)
    for g in range(_NCHUNK):
        cur = g % 2
        nxt = (g + 1) % 2
        if g + 1 < _NCHUNK:
            if g >= 1:
                pltpu.make_async_copy(
                    bufs[nxt], out_hbm.at[pl.ds(0, _C)], sss[nxt]).wait()
            start_gather(g + 1, nxt)
        wait_gather(cur)
        scale_chunk(bufs[cur])
        pltpu.async_copy(bufs[cur], out_hbm.at[pl.ds(base + g * _C, _C)],
                         sss[cur])

    for b in ((_NCHUNK - 1) % 2, _NCHUNK % 2):
        pltpu.make_async_copy(bufs[b], out_hbm.at[pl.ds(0, _C)],
                              sss[b]).wait()


def kernel(x, table):
    out = _sc_embed(x.reshape(_XROWS, 128), table)
    return out.reshape(x.shape + (_DIM,))
